# Initial kernel scaffold; baseline (speedup 1.0000x reference)
import functools

import jax
import jax.numpy as jnp
from jax import lax
from jax.experimental import pallas as pl
from jax.experimental.pallas import tpu as pltpu
from jax.experimental.pallas import tpu_sc as plsc

BATCH = 16384
SEQ = 50
DIM = 32
N_ROWS = BATCH * SEQ
NC, NS = 2, 16
NW = NC * NS
PER_W = N_ROWS // NW
IDX_W = 128
CHUNK = 1024
SUB = CHUNK // IDX_W
N_CHUNKS = PER_W // CHUNK


def _gather_body(idx_hbm, table_hbm, out_hbm, idx_v, rows_v, sem):
    wid = lax.axis_index("s") * NC + lax.axis_index("c")
    base = wid * PER_W

    def chunk(i, _):
        off = base + i * CHUNK
        pltpu.sync_copy(idx_hbm.at[pl.ds(off // IDX_W, SUB)], idx_v)
        copies = []
        for j in range(SUB):
            copies.append(
                pltpu.async_copy(
                    table_hbm.at[idx_v.at[j]],
                    rows_v.at[pl.ds(j * IDX_W, IDX_W)],
                    sem,
                )
            )
        for c in copies:
            c.wait()
        pltpu.sync_copy(rows_v, out_hbm.at[pl.ds(off, CHUNK)])
        return 0

    lax.fori_loop(0, N_CHUNKS, chunk, 0)


_gather = functools.partial(
    pl.kernel,
    mesh=plsc.VectorSubcoreMesh(core_axis_name="c", subcore_axis_name="s"),
    out_type=jax.ShapeDtypeStruct((N_ROWS, DIM), jnp.float32),
    scratch_types=[
        pltpu.VMEM((SUB, IDX_W), jnp.int32),
        pltpu.VMEM((CHUNK, DIM), jnp.float32),
        pltpu.SemaphoreType.DMA,
    ],
)(_gather_body)


def _tr_body(x_ref, o_ref):
    o_ref[...] = jnp.transpose(x_ref[...], (0, 2, 1))


_NB = 256

_transpose = pl.pallas_call(
    _tr_body,
    grid=(BATCH // _NB,),
    in_specs=[pl.BlockSpec((_NB, SEQ, DIM), lambda i: (i, 0, 0))],
    out_specs=pl.BlockSpec((_NB, DIM, SEQ), lambda i: (i, 0, 0)),
    out_shape=jax.ShapeDtypeStruct((BATCH, DIM, SEQ), jnp.float32),
)


def kernel(inputs, z):
    idx2d = jnp.reshape(inputs, (N_ROWS // IDX_W, IDX_W)).astype(jnp.int32)
    gathered = _gather(idx2d, z)
    return _transpose(jnp.reshape(gathered, (BATCH, SEQ, DIM)))


# speedup vs baseline: 1.4398x; 1.4398x over previous
import functools

import jax
import jax.numpy as jnp
from jax import lax
from jax.experimental import pallas as pl
from jax.experimental.pallas import tpu as pltpu
from jax.experimental.pallas import tpu_sc as plsc

BATCH = 16384
SEQ = 50
DIM = 32
N_ROWS = BATCH * SEQ
NC, NS = 2, 16
NW = NC * NS
PER_W = N_ROWS // NW
IDX_W = 128
CHUNK = 1024
SUB = CHUNK // IDX_W
N_CHUNKS = PER_W // CHUNK


def _gather_body(idx_hbm, table_hbm, out_hbm, idx_v, rows_v, sem):
    wid = lax.axis_index("s") * NC + lax.axis_index("c")
    base = wid * PER_W

    def chunk(i, _):
        off = base + i * CHUNK
        idx_off = pl.multiple_of(off // IDX_W, 8)
        pltpu.sync_copy(idx_hbm.at[pl.ds(idx_off, SUB)], idx_v)
        copies = []
        for j in range(SUB):
            copies.append(
                pltpu.async_copy(
                    table_hbm.at[idx_v.at[j]],
                    rows_v.at[pl.ds(j * IDX_W, IDX_W)],
                    sem,
                )
            )
        for c in copies:
            c.wait()
        pltpu.sync_copy(rows_v, out_hbm.at[pl.ds(off, CHUNK)])
        return 0

    lax.fori_loop(0, N_CHUNKS, chunk, 0)


_gather = functools.partial(
    pl.kernel,
    mesh=plsc.VectorSubcoreMesh(core_axis_name="c", subcore_axis_name="s"),
    out_type=jax.ShapeDtypeStruct((N_ROWS, DIM), jnp.float32),
    scratch_types=[
        pltpu.VMEM((SUB, IDX_W), jnp.int32),
        pltpu.VMEM((CHUNK, DIM), jnp.float32),
        pltpu.SemaphoreType.DMA,
    ],
    compiler_params=pltpu.CompilerParams(use_tc_tiling_on_sc=False),
)(_gather_body)


def _tr_body(x_ref, o_ref):
    o_ref[...] = jnp.transpose(x_ref[...], (0, 2, 1))


_NB = 256

_transpose = pl.pallas_call(
    _tr_body,
    grid=(BATCH // _NB,),
    in_specs=[pl.BlockSpec((_NB, SEQ, DIM), lambda i: (i, 0, 0))],
    out_specs=pl.BlockSpec((_NB, DIM, SEQ), lambda i: (i, 0, 0)),
    out_shape=jax.ShapeDtypeStruct((BATCH, DIM, SEQ), jnp.float32),
)


def kernel(inputs, z):
    idx2d = jnp.reshape(inputs, (N_ROWS // IDX_W, IDX_W)).astype(jnp.int32)
    gathered = _gather(idx2d, z)
    return _transpose(jnp.reshape(gathered, (BATCH, SEQ, DIM)))


# speedup vs baseline: 1.9538x; 1.3570x over previous
import functools

import jax
import jax.numpy as jnp
from jax import lax
from jax.experimental import pallas as pl
from jax.experimental.pallas import tpu as pltpu
from jax.experimental.pallas import tpu_sc as plsc

BATCH = 16384
SEQ = 50
DIM = 32
N_ROWS = BATCH * SEQ
NC, NS = 2, 16
NW = NC * NS
PER_W = N_ROWS // NW
IDX_W = 128
CHUNK = 1024
SUB = CHUNK // IDX_W
N_CHUNKS = PER_W // CHUNK


def _gather_body(idx_hbm, table_hbm, out_hbm, idx_v, rows_v, sem):
    wid = lax.axis_index("s") * NC + lax.axis_index("c")
    base = wid * PER_W

    def chunk(i, _):
        off = base + i * CHUNK
        idx_off = pl.multiple_of(off // IDX_W, 8)
        pltpu.sync_copy(idx_hbm.at[pl.ds(idx_off, SUB)], idx_v)
        copies = []
        for j in range(SUB):
            copies.append(
                pltpu.async_copy(
                    table_hbm.at[idx_v.at[j]],
                    rows_v.at[pl.ds(j * IDX_W, IDX_W)],
                    sem,
                )
            )
        for c in copies:
            c.wait()
        pltpu.sync_copy(rows_v, out_hbm.at[pl.ds(off, CHUNK)])
        return 0

    lax.fori_loop(0, N_CHUNKS, chunk, 0)


_gather = functools.partial(
    pl.kernel,
    mesh=plsc.VectorSubcoreMesh(core_axis_name="c", subcore_axis_name="s"),
    out_type=jax.ShapeDtypeStruct((N_ROWS, DIM), jnp.float32),
    scratch_types=[
        pltpu.VMEM((SUB, IDX_W), jnp.int32),
        pltpu.VMEM((CHUNK, DIM), jnp.float32),
        pltpu.SemaphoreType.DMA,
    ],
    compiler_params=pltpu.CompilerParams(use_tc_tiling_on_sc=False),
)(_gather_body)


def _tr_body(x_ref, o_ref):
    o_ref[...] = jnp.transpose(x_ref[...], (0, 2, 1))


_NB = 128

_transpose = pl.pallas_call(
    _tr_body,
    grid=(BATCH // 4 // _NB,),
    in_specs=[pl.BlockSpec((_NB, SEQ, 4 * DIM), lambda i: (i, 0, 0))],
    out_specs=pl.BlockSpec((_NB, 4 * DIM, SEQ), lambda i: (i, 0, 0)),
    out_shape=jax.ShapeDtypeStruct((BATCH // 4, 4 * DIM, SEQ), jnp.float32),
)


def kernel(inputs, z):
    idx_perm = jnp.transpose(
        jnp.reshape(inputs, (BATCH // 4, 4, SEQ)), (0, 2, 1)
    )
    idx2d = jnp.reshape(idx_perm, (N_ROWS // IDX_W, IDX_W)).astype(jnp.int32)
    gathered = _gather(idx2d, z)
    out4 = _transpose(jnp.reshape(gathered, (BATCH // 4, SEQ, 4 * DIM)))
    return jnp.reshape(out4, (BATCH, DIM, SEQ))
